# baseline (speedup 1.0000x reference)
import functools
import jax
import jax.numpy as jnp
from jax import lax
from jax.experimental import pallas as pl
from jax.experimental.pallas import tpu as pltpu
from jax.experimental.pallas import tpu_sc as plsc

_B, _N, _K, _FB = 76, 151, 10, 5
_H, _DEPTH = 128, 3
_AF = 89
_NP = 160
_FBP = 16
_NW = 32
_MPW = 3
_R = _B * _NP
_BLK = 1216


def _sc_gather_table(tbl_hbm, idx_hbm, out_hbm, tbl_loc, idx_loc, out_loc,
                     b, width):
    pltpu.sync_copy(tbl_hbm.at[pl.ds(b * _NP * width, _NP * width)], tbl_loc)
    pltpu.sync_copy(idx_hbm.at[b], idx_loc)
    for g in range(_NP // 16):
        out_base = (lax.iota(jnp.int32, 16) + g * 16) * width
        rvw = [idx_loc[k, pl.ds(g * 16, 16)] * width for k in range(_K)]

        def fbody(f, _):
            acc = jnp.zeros((16,), jnp.float32)
            for k in range(_K):
                acc = acc + plsc.load_gather(tbl_loc, [rvw[k] + f])
            plsc.store_scatter(out_loc, [out_base + f], acc)
            return 0

        lax.fori_loop(0, width, fbody, 0)
    pltpu.sync_copy(out_loc, out_hbm.at[pl.ds(b * _NP * width, _NP * width)])


def _sc_body_atom(a_hbm, idxa_hbm, sa_hbm, tbl_loc, idx_loc, out_loc):
    wid = lax.axis_index("s") * 2 + lax.axis_index("c")
    for m in range(_MPW):
        b = wid + _NW * m

        @pl.when(b < _B)
        def _():
            _sc_gather_table(a_hbm, idxa_hbm, sa_hbm,
                             tbl_loc, idx_loc, out_loc, b, _H)


def _sc_body_atom_bond(a_hbm, bond_hbm, idxa_hbm, idxb_hbm, sa_hbm, sb_hbm,
                       tbl_loc, idx_loc, out_loc, btbl_loc, bidx_loc,
                       bout_loc):
    wid = lax.axis_index("s") * 2 + lax.axis_index("c")
    for m in range(_MPW):
        b = wid + _NW * m

        @pl.when(b < _B)
        def _():
            _sc_gather_table(a_hbm, idxa_hbm, sa_hbm,
                             tbl_loc, idx_loc, out_loc, b, _H)
            _sc_gather_table(bond_hbm, idxb_hbm, sb_hbm,
                             btbl_loc, bidx_loc, bout_loc, b, _FBP)


_mesh = plsc.VectorSubcoreMesh(core_axis_name="c", subcore_axis_name="s")
_sc_params = pltpu.CompilerParams(needs_layout_passes=False)

_sc_gather = pl.kernel(
    _sc_body_atom,
    mesh=_mesh,
    compiler_params=_sc_params,
    out_type=jax.ShapeDtypeStruct((_R * _H,), jnp.float32),
    scratch_types=[
        pltpu.VMEM((_NP * _H,), jnp.float32),
        pltpu.VMEM((_K, _NP), jnp.int32),
        pltpu.VMEM((_NP * _H,), jnp.float32),
    ],
)

_sc_gather_ab = pl.kernel(
    _sc_body_atom_bond,
    mesh=_mesh,
    compiler_params=_sc_params,
    out_type=(jax.ShapeDtypeStruct((_R * _H,), jnp.float32),
              jax.ShapeDtypeStruct((_R * _FBP,), jnp.float32)),
    scratch_types=[
        pltpu.VMEM((_NP * _H,), jnp.float32),
        pltpu.VMEM((_K, _NP), jnp.int32),
        pltpu.VMEM((_NP * _H,), jnp.float32),
        pltpu.VMEM((_NP * _FBP,), jnp.float32),
        pltpu.VMEM((_K, _NP), jnp.int32),
        pltpu.VMEM((_NP * _FBP,), jnp.float32),
    ],
)


def _tc_pre_body(x_ref, WaT_ref, out_ref):
    out_ref[...] = jnp.dot(x_ref[...], WaT_ref[...],
                           preferred_element_type=jnp.float32)


def _tc_step_body(a_ref, s_ref, bsum_ref, W2aT_ref, W2bT_ref, W1aT_ref,
                  W1bT_ref, bU1_ref, out_ref):
    f32 = jnp.float32
    pid = pl.program_id(0)
    bondpart = jnp.dot(bsum_ref[...], W2bT_ref[...], preferred_element_type=f32)
    nei = jnp.dot(s_ref[...], W2aT_ref[...], preferred_element_type=f32) + bondpart
    a_new = (jnp.dot(a_ref[...], W1aT_ref[...], preferred_element_type=f32)
             + jnp.dot(nei, W1bT_ref[...], preferred_element_type=f32)
             + bU1_ref[...])
    row = lax.broadcasted_iota(jnp.int32, (_BLK, _H), 0) + pid * _BLK
    valid = (lax.rem(row, _NP) < _N).astype(f32)
    out_ref[...] = a_new * valid


def _row_spec():
    return pl.BlockSpec((_BLK, _H), lambda i: (i, 0))


_tc_pre = functools.partial(
    pl.pallas_call, _tc_pre_body,
    grid=(_R // _BLK,),
    in_specs=[_row_spec(), pl.BlockSpec((_H, _H), lambda i: (0, 0))],
    out_specs=_row_spec(),
    out_shape=jax.ShapeDtypeStruct((_R, _H), jnp.float32),
)()

_tc_step = functools.partial(
    pl.pallas_call, _tc_step_body,
    grid=(_R // _BLK,),
    in_specs=[
        _row_spec(), _row_spec(),
        pl.BlockSpec((_BLK, _FBP), lambda i: (i, 0)),
        pl.BlockSpec((_H, _H), lambda i: (0, 0)),
        pl.BlockSpec((_FBP, _H), lambda i: (0, 0)),
        pl.BlockSpec((_H, _H), lambda i: (0, 0)),
        pl.BlockSpec((_H, _H), lambda i: (0, 0)),
        pl.BlockSpec((1, _H), lambda i: (0, 0)),
    ],
    out_specs=_row_spec(),
    out_shape=jax.ShapeDtypeStruct((_R, _H), jnp.float32),
)()


def kernel(input_atom, input_bond, atom_nei_idx, bond_nei_idx, num_nbs,
           W_atom, W_U2, b_U2, W_U1, b_U1):
    f32 = jnp.float32
    xp = jnp.zeros((_B, _NP, _H), f32)
    xp = xp.at[:, :_N, :_AF].set(input_atom)
    bond_tbl = jnp.zeros((_B, _NP, _FBP), f32)
    bond_tbl = bond_tbl.at[:, :_N, :_FB].set(input_bond)
    bond_tbl = bond_tbl.at[:, :_N, _FB].set(1.0)

    mask = jnp.arange(_K, dtype=jnp.int32)[None, None, :] < num_nbs[:, :, None]
    idxa = jnp.where(mask, atom_nei_idx, _NP - 1)
    idxb = jnp.where(mask, bond_nei_idx[..., 0], _NP - 1)
    pad = jnp.full((_B, _NP - _N, _K), _NP - 1, jnp.int32)
    idxa_t = jnp.transpose(jnp.concatenate([idxa, pad], 1), (0, 2, 1))
    idxb_t = jnp.transpose(jnp.concatenate([idxb, pad], 1), (0, 2, 1))

    WaT = jnp.zeros((_H, _H), f32).at[:_AF, :].set(W_atom.T)
    W2aT = W_U2[:, :_H].T
    W2b_aug = jnp.concatenate([W_U2[:, _H:], b_U2[:, None]], axis=1)
    W2bT = jnp.zeros((_FBP, _H), f32).at[:_FB + 1, :].set(W2b_aug.T)
    W1aT = W_U1[:, :_H].T
    W1bT = W_U1[:, _H:].T
    bU1 = b_U1[None, :]

    A = _tc_pre(xp.reshape(_R, _H), WaT)
    S_flat, bsum_flat = _sc_gather_ab(
        A.reshape(_R * _H), bond_tbl.reshape(_R * _FBP), idxa_t, idxb_t)
    bsum = bsum_flat.reshape(_R, _FBP)
    A = _tc_step(A, S_flat.reshape(_R, _H), bsum, W2aT, W2bT, W1aT, W1bT, bU1)
    for _ in range(_DEPTH - 1):
        S_flat = _sc_gather(A.reshape(_R * _H), idxa_t)
        A = _tc_step(A, S_flat.reshape(_R, _H), bsum,
                     W2aT, W2bT, W1aT, W1bT, bU1)
    return A.reshape(_B, _NP, _H)[:, :_N, :]


# speedup vs baseline: 1.3501x; 1.3501x over previous
import functools
import jax
import jax.numpy as jnp
from jax import lax
from jax.experimental import pallas as pl
from jax.experimental.pallas import tpu as pltpu
from jax.experimental.pallas import tpu_sc as plsc

_B, _N, _K, _FB = 76, 151, 10, 5
_H, _DEPTH = 128, 3
_AF = 89
_NP = 160
_FBP = 16
_NW = 32
_MPW = 3
_R = _B * _NP
_BLK = 1216


def _sc_gather_table(tbl_hbm, idx_hbm, out_hbm, tbl_loc, idx_loc, out_loc,
                     b, width):
    pltpu.sync_copy(tbl_hbm.at[pl.ds(b * _NP * width, _NP * width)], tbl_loc)
    pltpu.sync_copy(idx_hbm.at[b], idx_loc)
    un = 8
    for g in range(_NP // 16):
        out_base = (lax.iota(jnp.int32, 16) + g * 16) * width
        rvw = [idx_loc[k, pl.ds(g * 16, 16)] * width for k in range(_K)]

        def fbody(i, _):
            f0 = i * un
            accs = [jnp.zeros((16,), jnp.float32) for _ in range(un)]
            for k in range(_K):
                a0 = rvw[k] + f0
                for d in range(un):
                    accs[d] = accs[d] + plsc.load_gather(tbl_loc, [a0 + d])
            ob = out_base + f0
            for d in range(un):
                plsc.store_scatter(out_loc, [ob + d], accs[d])
            return 0

        lax.fori_loop(0, width // un, fbody, 0)
    pltpu.sync_copy(out_loc, out_hbm.at[pl.ds(b * _NP * width, _NP * width)])


def _sc_body_atom(a_hbm, idxa_hbm, sa_hbm, tbl_loc, idx_loc, out_loc):
    wid = lax.axis_index("s") * 2 + lax.axis_index("c")
    for m in range(_MPW):
        b = wid + _NW * m

        @pl.when(b < _B)
        def _():
            _sc_gather_table(a_hbm, idxa_hbm, sa_hbm,
                             tbl_loc, idx_loc, out_loc, b, _H)


def _sc_body_atom_bond(a_hbm, bond_hbm, idxa_hbm, idxb_hbm, sa_hbm, sb_hbm,
                       tbl_loc, idx_loc, out_loc, btbl_loc, bidx_loc,
                       bout_loc):
    wid = lax.axis_index("s") * 2 + lax.axis_index("c")
    for m in range(_MPW):
        b = wid + _NW * m

        @pl.when(b < _B)
        def _():
            _sc_gather_table(a_hbm, idxa_hbm, sa_hbm,
                             tbl_loc, idx_loc, out_loc, b, _H)
            _sc_gather_table(bond_hbm, idxb_hbm, sb_hbm,
                             btbl_loc, bidx_loc, bout_loc, b, _FBP)


_mesh = plsc.VectorSubcoreMesh(core_axis_name="c", subcore_axis_name="s")
_sc_params = pltpu.CompilerParams(needs_layout_passes=False)

_sc_gather = pl.kernel(
    _sc_body_atom,
    mesh=_mesh,
    compiler_params=_sc_params,
    out_type=jax.ShapeDtypeStruct((_R * _H,), jnp.float32),
    scratch_types=[
        pltpu.VMEM((_NP * _H,), jnp.float32),
        pltpu.VMEM((_K, _NP), jnp.int32),
        pltpu.VMEM((_NP * _H,), jnp.float32),
    ],
)

_sc_gather_ab = pl.kernel(
    _sc_body_atom_bond,
    mesh=_mesh,
    compiler_params=_sc_params,
    out_type=(jax.ShapeDtypeStruct((_R * _H,), jnp.float32),
              jax.ShapeDtypeStruct((_R * _FBP,), jnp.float32)),
    scratch_types=[
        pltpu.VMEM((_NP * _H,), jnp.float32),
        pltpu.VMEM((_K, _NP), jnp.int32),
        pltpu.VMEM((_NP * _H,), jnp.float32),
        pltpu.VMEM((_NP * _FBP,), jnp.float32),
        pltpu.VMEM((_K, _NP), jnp.int32),
        pltpu.VMEM((_NP * _FBP,), jnp.float32),
    ],
)


def _tc_pre_body(x_ref, WaT_ref, out_ref):
    out_ref[...] = jnp.dot(x_ref[...], WaT_ref[...],
                           preferred_element_type=jnp.float32)


def _tc_step_body(a_ref, s_ref, bsum_ref, W2aT_ref, W2bT_ref, W1aT_ref,
                  W1bT_ref, bU1_ref, out_ref):
    f32 = jnp.float32
    pid = pl.program_id(0)
    bondpart = jnp.dot(bsum_ref[...], W2bT_ref[...], preferred_element_type=f32)
    nei = jnp.dot(s_ref[...], W2aT_ref[...], preferred_element_type=f32) + bondpart
    a_new = (jnp.dot(a_ref[...], W1aT_ref[...], preferred_element_type=f32)
             + jnp.dot(nei, W1bT_ref[...], preferred_element_type=f32)
             + bU1_ref[...])
    row = lax.broadcasted_iota(jnp.int32, (_BLK, _H), 0) + pid * _BLK
    valid = (lax.rem(row, _NP) < _N).astype(f32)
    out_ref[...] = a_new * valid


def _row_spec():
    return pl.BlockSpec((_BLK, _H), lambda i: (i, 0))


_tc_pre = functools.partial(
    pl.pallas_call, _tc_pre_body,
    grid=(_R // _BLK,),
    in_specs=[_row_spec(), pl.BlockSpec((_H, _H), lambda i: (0, 0))],
    out_specs=_row_spec(),
    out_shape=jax.ShapeDtypeStruct((_R, _H), jnp.float32),
)()

_tc_step = functools.partial(
    pl.pallas_call, _tc_step_body,
    grid=(_R // _BLK,),
    in_specs=[
        _row_spec(), _row_spec(),
        pl.BlockSpec((_BLK, _FBP), lambda i: (i, 0)),
        pl.BlockSpec((_H, _H), lambda i: (0, 0)),
        pl.BlockSpec((_FBP, _H), lambda i: (0, 0)),
        pl.BlockSpec((_H, _H), lambda i: (0, 0)),
        pl.BlockSpec((_H, _H), lambda i: (0, 0)),
        pl.BlockSpec((1, _H), lambda i: (0, 0)),
    ],
    out_specs=_row_spec(),
    out_shape=jax.ShapeDtypeStruct((_R, _H), jnp.float32),
)()


def kernel(input_atom, input_bond, atom_nei_idx, bond_nei_idx, num_nbs,
           W_atom, W_U2, b_U2, W_U1, b_U1):
    f32 = jnp.float32
    xp = jnp.zeros((_B, _NP, _H), f32)
    xp = xp.at[:, :_N, :_AF].set(input_atom)
    bond_tbl = jnp.zeros((_B, _NP, _FBP), f32)
    bond_tbl = bond_tbl.at[:, :_N, :_FB].set(input_bond)
    bond_tbl = bond_tbl.at[:, :_N, _FB].set(1.0)

    mask = jnp.arange(_K, dtype=jnp.int32)[None, None, :] < num_nbs[:, :, None]
    idxa = jnp.where(mask, atom_nei_idx, _NP - 1)
    idxb = jnp.where(mask, bond_nei_idx[..., 0], _NP - 1)
    pad = jnp.full((_B, _NP - _N, _K), _NP - 1, jnp.int32)
    idxa_t = jnp.transpose(jnp.concatenate([idxa, pad], 1), (0, 2, 1))
    idxb_t = jnp.transpose(jnp.concatenate([idxb, pad], 1), (0, 2, 1))

    WaT = jnp.zeros((_H, _H), f32).at[:_AF, :].set(W_atom.T)
    W2aT = W_U2[:, :_H].T
    W2b_aug = jnp.concatenate([W_U2[:, _H:], b_U2[:, None]], axis=1)
    W2bT = jnp.zeros((_FBP, _H), f32).at[:_FB + 1, :].set(W2b_aug.T)
    W1aT = W_U1[:, :_H].T
    W1bT = W_U1[:, _H:].T
    bU1 = b_U1[None, :]

    A = _tc_pre(xp.reshape(_R, _H), WaT)
    S_flat, bsum_flat = _sc_gather_ab(
        A.reshape(_R * _H), bond_tbl.reshape(_R * _FBP), idxa_t, idxb_t)
    bsum = bsum_flat.reshape(_R, _FBP)
    A = _tc_step(A, S_flat.reshape(_R, _H), bsum, W2aT, W2bT, W1aT, W1bT, bU1)
    for _ in range(_DEPTH - 1):
        S_flat = _sc_gather(A.reshape(_R * _H), idxa_t)
        A = _tc_step(A, S_flat.reshape(_R, _H), bsum,
                     W2aT, W2bT, W1aT, W1bT, bU1)
    return A.reshape(_B, _NP, _H)[:, :_N, :]
